# Initial kernel scaffold; baseline (speedup 1.0000x reference)
import jax
import jax.numpy as jnp
from jax.experimental import pallas as pl


def kernel(logits, target):
    raise NotImplementedError("write your pallas kernel here")


# speedup vs baseline: 1.9373x; 1.9373x over previous
import functools

import jax
import jax.numpy as jnp
from jax import lax
from jax.experimental import pallas as pl
from jax.experimental.pallas import tpu as pltpu
from jax.experimental.pallas import tpu_sc as plsc

_N = 65536
_C = 1000
_CPAD = 1024
_R = 512
_NB = _N // _R
_NC = 2
_NS = 16
_NW = _NC * _NS
_CHUNK = _N // _NW
_STEPS = _CHUNK // 16


def _pt_body(l_ref, t_ref, pt_ref, lg_ref):
    l = l_ref[...]
    t = t_ref[...]
    m = jnp.max(l, axis=1, keepdims=True)
    e = jnp.exp(l - m)
    s = jnp.sum(e, axis=1, keepdims=True)
    cols = lax.broadcasted_iota(jnp.int32, (_R, _C), 1)
    et = jnp.sum(jnp.where(cols == t, e, 0.0), axis=1, keepdims=True)
    pt = et / s
    pt_ref[...] = pt
    lg_ref[...] = jnp.log(pt + 1e-6)


def _pt_pass(logits, t2d, interpret=False):
    return pl.pallas_call(
        _pt_body,
        grid=(_NB,),
        in_specs=[
            pl.BlockSpec((_R, _C), lambda i: (i, 0)),
            pl.BlockSpec((_R, 1), lambda i: (i, 0)),
        ],
        out_specs=[
            pl.BlockSpec((_R, 1), lambda i: (i, 0)),
            pl.BlockSpec((_R, 1), lambda i: (i, 0)),
        ],
        out_shape=[
            jax.ShapeDtypeStruct((_N, 1), jnp.float32),
            jax.ShapeDtypeStruct((_N, 1), jnp.float32),
        ],
        interpret=interpret,
    )(logits, t2d)


def _seg_body(pt_hbm, lg_hbm, t_hbm, cnt_out, s1_out, s2_out, sl_out,
              pt_v, lg_v, t_v, cnt_v, s1_v, s2_v, sl_v):
    wid = lax.axis_index("s") * _NC + lax.axis_index("c")
    base = wid * _CHUNK
    pltpu.sync_copy(pt_hbm.at[pl.ds(base, _CHUNK)], pt_v)
    pltpu.sync_copy(lg_hbm.at[pl.ds(base, _CHUNK)], lg_v)
    pltpu.sync_copy(t_hbm.at[pl.ds(base, _CHUNK)], t_v)

    zeros16 = jnp.zeros((16,), jnp.float32)

    def zbody(i, carry):
        sl16 = pl.ds(i * 16, 16)
        cnt_v[sl16] = zeros16
        s1_v[sl16] = zeros16
        s2_v[sl16] = zeros16
        sl_v[sl16] = zeros16
        return carry

    lax.fori_loop(0, _CPAD // 16, zbody, 0)

    ones16 = jnp.full((16,), 1.0, jnp.float32)

    def body(i, carry):
        sl16 = pl.ds(i * 16, 16)
        t16 = t_v[sl16]
        p16 = pt_v[sl16]
        g16 = lg_v[sl16]
        plsc.addupdate_scatter(cnt_v, [t16], ones16)
        plsc.addupdate_scatter(s1_v, [t16], p16)
        plsc.addupdate_scatter(s2_v, [t16], p16 * p16)
        plsc.addupdate_scatter(sl_v, [t16], g16)
        return carry

    lax.fori_loop(0, _STEPS, body, 0)

    pltpu.sync_copy(cnt_v, cnt_out.at[wid])
    pltpu.sync_copy(s1_v, s1_out.at[wid])
    pltpu.sync_copy(s2_v, s2_out.at[wid])
    pltpu.sync_copy(sl_v, sl_out.at[wid])


@functools.cache
def _make_seg_call():
    return functools.partial(
        pl.kernel,
        mesh=plsc.VectorSubcoreMesh(core_axis_name="c", subcore_axis_name="s"),
        compiler_params=pltpu.CompilerParams(needs_layout_passes=False),
        out_type=[jax.ShapeDtypeStruct((_NW, _CPAD), jnp.float32)] * 4,
        scratch_types=[
            pltpu.VMEM((_CHUNK,), jnp.float32),
            pltpu.VMEM((_CHUNK,), jnp.float32),
            pltpu.VMEM((_CHUNK,), jnp.int32),
            pltpu.VMEM((_CPAD,), jnp.float32),
            pltpu.VMEM((_CPAD,), jnp.float32),
            pltpu.VMEM((_CPAD,), jnp.float32),
            pltpu.VMEM((_CPAD,), jnp.float32),
        ],
    )(_seg_body)


def _final_body(cnt_ref, s1_ref, s2_ref, sl_ref, out_ref):
    cnt = jnp.sum(cnt_ref[...], axis=0, keepdims=True)
    s1 = jnp.sum(s1_ref[...], axis=0, keepdims=True)
    s2 = jnp.sum(s2_ref[...], axis=0, keepdims=True)
    sl = jnp.sum(sl_ref[...], axis=0, keepdims=True)
    idx = lax.broadcasted_iota(jnp.int32, (1, _CPAD), 1)
    valid = idx < _C
    nz = cnt > 0.0
    csafe = jnp.where(nz, cnt, 1.0)
    cmax = jnp.max(cnt)
    alpha = jnp.where(nz, jnp.log(cmax / csafe) + 1.0, 0.0)
    p_avg1 = jnp.where(nz, -sl / csafe, 1.0)
    loss1 = p_avg1 * alpha
    loss1_mean = jnp.sum(jnp.where(valid, loss1, 0.0)) / _C
    mean = jnp.where(nz, s1 / csafe, 1.0)
    gt1 = cnt > 1.0
    denom = jnp.where(gt1, cnt - 1.0, 1.0)
    var = (s2 - cnt * mean * mean) / denom
    std = jnp.where(gt1, jnp.sqrt(jnp.maximum(var, 1e-12)), 0.0)
    am = jnp.max(jnp.where(valid, alpha, -1e30))
    ea = jnp.where(valid, jnp.exp(alpha - am), 0.0)
    asm = ea / jnp.sum(ea)
    loss2_c = std / mean * asm
    loss2_mean = jnp.sum(cnt * loss2_c) / _N
    out_ref[...] = jnp.full((1, 1), loss1_mean + loss2_mean, jnp.float32)


def _final_pass(cnt_p, s1_p, s2_p, sl_p, interpret=False):
    return pl.pallas_call(
        _final_body,
        out_shape=jax.ShapeDtypeStruct((1, 1), jnp.float32),
        interpret=interpret,
    )(cnt_p, s1_p, s2_p, sl_p)


@jax.jit
def kernel(logits, target):
    logits = logits.reshape(-1, _C)
    t = target.reshape(-1).astype(jnp.int32)
    pt2d, lg2d = _pt_pass(logits, t.reshape(-1, 1))
    pt = pt2d.reshape(-1)
    lg = lg2d.reshape(-1)
    cnt_p, s1_p, s2_p, sl_p = _make_seg_call()(pt, lg, t)
    out = _final_pass(cnt_p, s1_p, s2_p, sl_p)
    return out[0, 0]


# speedup vs baseline: 2.1186x; 1.0936x over previous
import functools

import jax
import jax.numpy as jnp
from jax import lax
from jax.experimental import pallas as pl
from jax.experimental.pallas import tpu as pltpu
from jax.experimental.pallas import tpu_sc as plsc

_N = 65536
_C = 1000
_CPAD = 1024
_R = 1024
_NB = _N // _R
_NC = 2
_NS = 16
_NW = _NC * _NS
_CHUNK = _N // _NW
_STEPS = _CHUNK // 16


def _pt_body(l_ref, t_ref, pt_ref, lg_ref):
    l = l_ref[...]
    t = t_ref[...]
    m = jnp.max(l, axis=1, keepdims=True)
    e = jnp.exp(l - m)
    s = jnp.sum(e, axis=1, keepdims=True)
    cols = lax.broadcasted_iota(jnp.int32, (_R, _C), 1)
    et = jnp.sum(jnp.where(cols == t, e, 0.0), axis=1, keepdims=True)
    pt = et / s
    pt_ref[...] = pt
    lg_ref[...] = jnp.log(pt + 1e-6)


def _pt_pass(logits, t2d, interpret=False):
    return pl.pallas_call(
        _pt_body,
        grid=(_NB,),
        in_specs=[
            pl.BlockSpec((_R, _C), lambda i: (i, 0)),
            pl.BlockSpec((_R, 1), lambda i: (i, 0)),
        ],
        out_specs=[
            pl.BlockSpec((_R, 1), lambda i: (i, 0)),
            pl.BlockSpec((_R, 1), lambda i: (i, 0)),
        ],
        out_shape=[
            jax.ShapeDtypeStruct((_N, 1), jnp.float32),
            jax.ShapeDtypeStruct((_N, 1), jnp.float32),
        ],
        interpret=interpret,
    )(logits, t2d)


def _seg_body(pt_hbm, lg_hbm, t_hbm, cnt_out, s1_out, s2_out, sl_out,
              pt_v, lg_v, t_v, cnt_v, s1_v, s2_v, sl_v):
    wid = lax.axis_index("s") * _NC + lax.axis_index("c")
    base = wid * _CHUNK
    pltpu.sync_copy(pt_hbm.at[pl.ds(base, _CHUNK)], pt_v)
    pltpu.sync_copy(lg_hbm.at[pl.ds(base, _CHUNK)], lg_v)
    pltpu.sync_copy(t_hbm.at[pl.ds(base, _CHUNK)], t_v)

    zeros16 = jnp.zeros((16,), jnp.float32)

    def zbody(i, carry):
        sl16 = pl.ds(i * 16, 16)
        cnt_v[sl16] = zeros16
        s1_v[sl16] = zeros16
        s2_v[sl16] = zeros16
        sl_v[sl16] = zeros16
        return carry

    lax.fori_loop(0, _CPAD // 16, zbody, 0)

    ones16 = jnp.full((16,), 1.0, jnp.float32)

    def body(i, carry):
        sl16 = pl.ds(i * 16, 16)
        t16 = t_v[sl16]
        p16 = pt_v[sl16]
        g16 = lg_v[sl16]
        plsc.addupdate_scatter(cnt_v, [t16], ones16)
        plsc.addupdate_scatter(s1_v, [t16], p16)
        plsc.addupdate_scatter(s2_v, [t16], p16 * p16)
        plsc.addupdate_scatter(sl_v, [t16], g16)
        return carry

    lax.fori_loop(0, _STEPS, body, 0)

    pltpu.sync_copy(cnt_v, cnt_out.at[wid])
    pltpu.sync_copy(s1_v, s1_out.at[wid])
    pltpu.sync_copy(s2_v, s2_out.at[wid])
    pltpu.sync_copy(sl_v, sl_out.at[wid])


@functools.cache
def _make_seg_call():
    return functools.partial(
        pl.kernel,
        mesh=plsc.VectorSubcoreMesh(core_axis_name="c", subcore_axis_name="s"),
        compiler_params=pltpu.CompilerParams(needs_layout_passes=False),
        out_type=[jax.ShapeDtypeStruct((_NW, _CPAD), jnp.float32)] * 4,
        scratch_types=[
            pltpu.VMEM((_CHUNK,), jnp.float32),
            pltpu.VMEM((_CHUNK,), jnp.float32),
            pltpu.VMEM((_CHUNK,), jnp.int32),
            pltpu.VMEM((_CPAD,), jnp.float32),
            pltpu.VMEM((_CPAD,), jnp.float32),
            pltpu.VMEM((_CPAD,), jnp.float32),
            pltpu.VMEM((_CPAD,), jnp.float32),
        ],
    )(_seg_body)


def _final_body(cnt_ref, s1_ref, s2_ref, sl_ref, out_ref):
    cnt = jnp.sum(cnt_ref[...], axis=0, keepdims=True)
    s1 = jnp.sum(s1_ref[...], axis=0, keepdims=True)
    s2 = jnp.sum(s2_ref[...], axis=0, keepdims=True)
    sl = jnp.sum(sl_ref[...], axis=0, keepdims=True)
    idx = lax.broadcasted_iota(jnp.int32, (1, _CPAD), 1)
    valid = idx < _C
    nz = cnt > 0.0
    csafe = jnp.where(nz, cnt, 1.0)
    cmax = jnp.max(cnt)
    alpha = jnp.where(nz, jnp.log(cmax / csafe) + 1.0, 0.0)
    p_avg1 = jnp.where(nz, -sl / csafe, 1.0)
    loss1 = p_avg1 * alpha
    loss1_mean = jnp.sum(jnp.where(valid, loss1, 0.0)) / _C
    mean = jnp.where(nz, s1 / csafe, 1.0)
    gt1 = cnt > 1.0
    denom = jnp.where(gt1, cnt - 1.0, 1.0)
    var = (s2 - cnt * mean * mean) / denom
    std = jnp.where(gt1, jnp.sqrt(jnp.maximum(var, 1e-12)), 0.0)
    am = jnp.max(jnp.where(valid, alpha, -1e30))
    ea = jnp.where(valid, jnp.exp(alpha - am), 0.0)
    asm = ea / jnp.sum(ea)
    loss2_c = std / mean * asm
    loss2_mean = jnp.sum(cnt * loss2_c) / _N
    out_ref[...] = jnp.full((1, 1), loss1_mean + loss2_mean, jnp.float32)


def _final_pass(cnt_p, s1_p, s2_p, sl_p, interpret=False):
    return pl.pallas_call(
        _final_body,
        out_shape=jax.ShapeDtypeStruct((1, 1), jnp.float32),
        interpret=interpret,
    )(cnt_p, s1_p, s2_p, sl_p)


@jax.jit
def kernel(logits, target):
    logits = logits.reshape(-1, _C)
    t = target.reshape(-1).astype(jnp.int32)
    pt2d, lg2d = _pt_pass(logits, t.reshape(-1, 1))
    pt = pt2d.reshape(-1)
    lg = lg2d.reshape(-1)
    cnt_p, s1_p, s2_p, sl_p = _make_seg_call()(pt, lg, t)
    out = _final_pass(cnt_p, s1_p, s2_p, sl_p)
    return out[0, 0]


# speedup vs baseline: 2.2023x; 1.0395x over previous
import functools

import jax
import jax.numpy as jnp
from jax import lax
from jax.experimental import pallas as pl
from jax.experimental.pallas import tpu as pltpu
from jax.experimental.pallas import tpu_sc as plsc

_N = 65536
_C = 1000
_CPAD = 1024
_R = 2048
_NB = _N // _R
_NC = 2
_NS = 16
_NW = _NC * _NS
_CHUNK = _N // _NW
_STEPS = _CHUNK // 16


def _pt_body(l_ref, t_ref, pt_ref, lg_ref):
    l = l_ref[...]
    t = t_ref[...]
    m = jnp.max(l, axis=1, keepdims=True)
    e = jnp.exp(l - m)
    s = jnp.sum(e, axis=1, keepdims=True)
    cols = lax.broadcasted_iota(jnp.int32, (_R, _C), 1)
    et = jnp.sum(jnp.where(cols == t, e, 0.0), axis=1, keepdims=True)
    pt = et / s
    pt_ref[...] = pt
    lg_ref[...] = jnp.log(pt + 1e-6)


def _pt_pass(logits, t2d, interpret=False):
    return pl.pallas_call(
        _pt_body,
        grid=(_NB,),
        in_specs=[
            pl.BlockSpec((_R, _C), lambda i: (i, 0)),
            pl.BlockSpec((_R, 1), lambda i: (i, 0)),
        ],
        out_specs=[
            pl.BlockSpec((_R, 1), lambda i: (i, 0)),
            pl.BlockSpec((_R, 1), lambda i: (i, 0)),
        ],
        out_shape=[
            jax.ShapeDtypeStruct((_N, 1), jnp.float32),
            jax.ShapeDtypeStruct((_N, 1), jnp.float32),
        ],
        interpret=interpret,
    )(logits, t2d)


def _seg_body(pt_hbm, lg_hbm, t_hbm, cnt_out, s1_out, s2_out, sl_out,
              pt_v, lg_v, t_v, cnt_v, s1_v, s2_v, sl_v):
    wid = lax.axis_index("s") * _NC + lax.axis_index("c")
    base = wid * _CHUNK
    pltpu.sync_copy(pt_hbm.at[pl.ds(base, _CHUNK)], pt_v)
    pltpu.sync_copy(lg_hbm.at[pl.ds(base, _CHUNK)], lg_v)
    pltpu.sync_copy(t_hbm.at[pl.ds(base, _CHUNK)], t_v)

    zeros16 = jnp.zeros((16,), jnp.float32)

    def zbody(i, carry):
        sl16 = pl.ds(i * 16, 16)
        cnt_v[sl16] = zeros16
        s1_v[sl16] = zeros16
        s2_v[sl16] = zeros16
        sl_v[sl16] = zeros16
        return carry

    lax.fori_loop(0, _CPAD // 16, zbody, 0)

    ones16 = jnp.full((16,), 1.0, jnp.float32)

    def body(i, carry):
        sl16 = pl.ds(i * 16, 16)
        t16 = t_v[sl16]
        p16 = pt_v[sl16]
        g16 = lg_v[sl16]
        plsc.addupdate_scatter(cnt_v, [t16], ones16)
        plsc.addupdate_scatter(s1_v, [t16], p16)
        plsc.addupdate_scatter(s2_v, [t16], p16 * p16)
        plsc.addupdate_scatter(sl_v, [t16], g16)
        return carry

    lax.fori_loop(0, _STEPS, body, 0)

    pltpu.sync_copy(cnt_v, cnt_out.at[wid])
    pltpu.sync_copy(s1_v, s1_out.at[wid])
    pltpu.sync_copy(s2_v, s2_out.at[wid])
    pltpu.sync_copy(sl_v, sl_out.at[wid])


@functools.cache
def _make_seg_call():
    return functools.partial(
        pl.kernel,
        mesh=plsc.VectorSubcoreMesh(core_axis_name="c", subcore_axis_name="s"),
        compiler_params=pltpu.CompilerParams(needs_layout_passes=False),
        out_type=[jax.ShapeDtypeStruct((_NW, _CPAD), jnp.float32)] * 4,
        scratch_types=[
            pltpu.VMEM((_CHUNK,), jnp.float32),
            pltpu.VMEM((_CHUNK,), jnp.float32),
            pltpu.VMEM((_CHUNK,), jnp.int32),
            pltpu.VMEM((_CPAD,), jnp.float32),
            pltpu.VMEM((_CPAD,), jnp.float32),
            pltpu.VMEM((_CPAD,), jnp.float32),
            pltpu.VMEM((_CPAD,), jnp.float32),
        ],
    )(_seg_body)


def _final_body(cnt_ref, s1_ref, s2_ref, sl_ref, out_ref):
    cnt = jnp.sum(cnt_ref[...], axis=0, keepdims=True)
    s1 = jnp.sum(s1_ref[...], axis=0, keepdims=True)
    s2 = jnp.sum(s2_ref[...], axis=0, keepdims=True)
    sl = jnp.sum(sl_ref[...], axis=0, keepdims=True)
    idx = lax.broadcasted_iota(jnp.int32, (1, _CPAD), 1)
    valid = idx < _C
    nz = cnt > 0.0
    csafe = jnp.where(nz, cnt, 1.0)
    cmax = jnp.max(cnt)
    alpha = jnp.where(nz, jnp.log(cmax / csafe) + 1.0, 0.0)
    p_avg1 = jnp.where(nz, -sl / csafe, 1.0)
    loss1 = p_avg1 * alpha
    loss1_mean = jnp.sum(jnp.where(valid, loss1, 0.0)) / _C
    mean = jnp.where(nz, s1 / csafe, 1.0)
    gt1 = cnt > 1.0
    denom = jnp.where(gt1, cnt - 1.0, 1.0)
    var = (s2 - cnt * mean * mean) / denom
    std = jnp.where(gt1, jnp.sqrt(jnp.maximum(var, 1e-12)), 0.0)
    am = jnp.max(jnp.where(valid, alpha, -1e30))
    ea = jnp.where(valid, jnp.exp(alpha - am), 0.0)
    asm = ea / jnp.sum(ea)
    loss2_c = std / mean * asm
    loss2_mean = jnp.sum(cnt * loss2_c) / _N
    out_ref[...] = jnp.full((1, 1), loss1_mean + loss2_mean, jnp.float32)


def _final_pass(cnt_p, s1_p, s2_p, sl_p, interpret=False):
    return pl.pallas_call(
        _final_body,
        out_shape=jax.ShapeDtypeStruct((1, 1), jnp.float32),
        interpret=interpret,
    )(cnt_p, s1_p, s2_p, sl_p)


@jax.jit
def kernel(logits, target):
    logits = logits.reshape(-1, _C)
    t = target.reshape(-1).astype(jnp.int32)
    pt2d, lg2d = _pt_pass(logits, t.reshape(-1, 1))
    pt = pt2d.reshape(-1)
    lg = lg2d.reshape(-1)
    cnt_p, s1_p, s2_p, sl_p = _make_seg_call()(pt, lg, t)
    out = _final_pass(cnt_p, s1_p, s2_p, sl_p)
    return out[0, 0]


# speedup vs baseline: 6.6799x; 3.0331x over previous
import functools

import jax
import jax.numpy as jnp
from jax import lax
from jax.experimental import pallas as pl
from jax.experimental.pallas import tpu as pltpu
from jax.experimental.pallas import tpu_sc as plsc

_N = 65536
_H = _N // 2
_C = 1000
_CPAD = 1024
_S = 4096
_NB2 = _H // _S
_NC = 2
_NS = 16
_NW = _NC * _NS
_CHUNK = _H // _NW
_STEPS = _CHUNK // 16


def _pt_body(l_ref, t_ref, pt_ref, lg_ref):
    l = l_ref[...]
    t = t_ref[...]
    m = jnp.max(l, axis=0, keepdims=True)
    e = jnp.exp(l - m)
    s = jnp.sum(e, axis=0, keepdims=True)
    rows = lax.broadcasted_iota(jnp.int32, (_C, _S), 0)
    et = jnp.sum(jnp.where(rows == t, e, 0.0), axis=0, keepdims=True)
    pt = et / s
    pt_ref[...] = pt
    lg_ref[...] = jnp.log(pt + 1e-6)


def _pt_pass(lT, t2d, half, interpret=False):
    base = half * _NB2
    return pl.pallas_call(
        _pt_body,
        grid=(_NB2,),
        in_specs=[
            pl.BlockSpec((_C, _S), lambda i: (0, base + i)),
            pl.BlockSpec((1, _S), lambda i: (0, base + i)),
        ],
        out_specs=[
            pl.BlockSpec((1, _S), lambda i: (0, i)),
            pl.BlockSpec((1, _S), lambda i: (0, i)),
        ],
        out_shape=[
            jax.ShapeDtypeStruct((1, _H), jnp.float32),
            jax.ShapeDtypeStruct((1, _H), jnp.float32),
        ],
        compiler_params=pltpu.CompilerParams(
            vmem_limit_bytes=100 * 1024 * 1024,
        ),
        interpret=interpret,
    )(lT, t2d)


def _seg_body(pt_hbm, lg_hbm, t_hbm, cnt_out, s1_out, s2_out, sl_out,
              pt_v, lg_v, t_v, cnt_v, s1_v, s2_v, sl_v):
    wid = lax.axis_index("s") * _NC + lax.axis_index("c")
    base = wid * _CHUNK
    pltpu.sync_copy(pt_hbm.at[pl.ds(base, _CHUNK)], pt_v)
    pltpu.sync_copy(lg_hbm.at[pl.ds(base, _CHUNK)], lg_v)
    pltpu.sync_copy(t_hbm.at[pl.ds(base, _CHUNK)], t_v)

    zeros16 = jnp.zeros((16,), jnp.float32)

    def zbody(i, carry):
        sl16 = pl.ds(i * 16, 16)
        cnt_v[sl16] = zeros16
        s1_v[sl16] = zeros16
        s2_v[sl16] = zeros16
        sl_v[sl16] = zeros16
        return carry

    lax.fori_loop(0, _CPAD // 16, zbody, 0)

    ones16 = jnp.full((16,), 1.0, jnp.float32)

    def body(i, carry):
        sl16 = pl.ds(i * 16, 16)
        t16 = t_v[sl16]
        p16 = pt_v[sl16]
        g16 = lg_v[sl16]
        plsc.addupdate_scatter(cnt_v, [t16], ones16)
        plsc.addupdate_scatter(s1_v, [t16], p16)
        plsc.addupdate_scatter(s2_v, [t16], p16 * p16)
        plsc.addupdate_scatter(sl_v, [t16], g16)
        return carry

    lax.fori_loop(0, _STEPS, body, 0)

    pltpu.sync_copy(cnt_v, cnt_out.at[wid])
    pltpu.sync_copy(s1_v, s1_out.at[wid])
    pltpu.sync_copy(s2_v, s2_out.at[wid])
    pltpu.sync_copy(sl_v, sl_out.at[wid])


@functools.cache
def _make_seg_call():
    return functools.partial(
        pl.kernel,
        mesh=plsc.VectorSubcoreMesh(core_axis_name="c", subcore_axis_name="s"),
        compiler_params=pltpu.CompilerParams(needs_layout_passes=False),
        out_type=[jax.ShapeDtypeStruct((_NW, _CPAD), jnp.float32)] * 4,
        scratch_types=[
            pltpu.VMEM((_CHUNK,), jnp.float32),
            pltpu.VMEM((_CHUNK,), jnp.float32),
            pltpu.VMEM((_CHUNK,), jnp.int32),
            pltpu.VMEM((_CPAD,), jnp.float32),
            pltpu.VMEM((_CPAD,), jnp.float32),
            pltpu.VMEM((_CPAD,), jnp.float32),
            pltpu.VMEM((_CPAD,), jnp.float32),
        ],
    )(_seg_body)


def _final_body(cnt_a, s1_a, s2_a, sl_a, cnt_b, s1_b, s2_b, sl_b, out_ref):
    cnt = jnp.sum(cnt_a[...], axis=0, keepdims=True) + \
        jnp.sum(cnt_b[...], axis=0, keepdims=True)
    s1 = jnp.sum(s1_a[...], axis=0, keepdims=True) + \
        jnp.sum(s1_b[...], axis=0, keepdims=True)
    s2 = jnp.sum(s2_a[...], axis=0, keepdims=True) + \
        jnp.sum(s2_b[...], axis=0, keepdims=True)
    sl = jnp.sum(sl_a[...], axis=0, keepdims=True) + \
        jnp.sum(sl_b[...], axis=0, keepdims=True)
    idx = lax.broadcasted_iota(jnp.int32, (1, _CPAD), 1)
    valid = idx < _C
    nz = cnt > 0.0
    csafe = jnp.where(nz, cnt, 1.0)
    cmax = jnp.max(cnt)
    alpha = jnp.where(nz, jnp.log(cmax / csafe) + 1.0, 0.0)
    p_avg1 = jnp.where(nz, -sl / csafe, 1.0)
    loss1 = p_avg1 * alpha
    loss1_mean = jnp.sum(jnp.where(valid, loss1, 0.0)) / _C
    mean = jnp.where(nz, s1 / csafe, 1.0)
    gt1 = cnt > 1.0
    denom = jnp.where(gt1, cnt - 1.0, 1.0)
    var = (s2 - cnt * mean * mean) / denom
    std = jnp.where(gt1, jnp.sqrt(jnp.maximum(var, 1e-12)), 0.0)
    am = jnp.max(jnp.where(valid, alpha, -1e30))
    ea = jnp.where(valid, jnp.exp(alpha - am), 0.0)
    asm = ea / jnp.sum(ea)
    loss2_c = std / mean * asm
    loss2_mean = jnp.sum(cnt * loss2_c) / _N
    out_ref[...] = jnp.full((1, 1), loss1_mean + loss2_mean, jnp.float32)


def _final_pass(parts_a, parts_b, interpret=False):
    return pl.pallas_call(
        _final_body,
        out_shape=jax.ShapeDtypeStruct((1, 1), jnp.float32),
        interpret=interpret,
    )(*parts_a, *parts_b)


@jax.jit
def kernel(logits, target):
    logits = logits.reshape(-1, _C)
    t = target.reshape(-1).astype(jnp.int32)
    lT = logits.T
    t2d = t.reshape(1, -1)
    seg = _make_seg_call()
    pt_a, lg_a = _pt_pass(lT, t2d, 0)
    parts_a = seg(pt_a.reshape(-1), lg_a.reshape(-1), t[:_H])
    pt_b, lg_b = _pt_pass(lT, t2d, 1)
    parts_b = seg(pt_b.reshape(-1), lg_b.reshape(-1), t[_H:])
    out = _final_pass(parts_a, parts_b)
    return out[0, 0]


# speedup vs baseline: 7.1646x; 1.0726x over previous
import functools

import jax
import jax.numpy as jnp
from jax import lax
from jax.experimental import pallas as pl
from jax.experimental.pallas import tpu as pltpu
from jax.experimental.pallas import tpu_sc as plsc

_N = 65536
_C = 1000
_CPAD = 1024
_S = 4096
_NB = _N // _S
_NC = 2
_NS = 16
_NW = _NC * _NS
_CHUNK = _N // _NW
_STEPS = _CHUNK // 16


def _pt_body(l_ref, t_ref, pt_ref, lg_ref):
    l = l_ref[...]
    t = t_ref[...]
    m = jnp.max(l, axis=0, keepdims=True)
    s = jnp.sum(jnp.exp(l - m), axis=0, keepdims=True)
    rows = lax.broadcasted_iota(jnp.int32, (_C, _S), 0)
    lt = jnp.sum(jnp.where(rows == t, l, 0.0), axis=0, keepdims=True)
    pt = jnp.exp(lt - m) / s
    pt_ref[...] = pt
    lg_ref[...] = jnp.log(pt + 1e-6)


def _pt_pass(lT, t2d, interpret=False):
    return pl.pallas_call(
        _pt_body,
        grid=(_NB,),
        in_specs=[
            pl.BlockSpec((_C, _S), lambda i: (0, i)),
            pl.BlockSpec((1, _S), lambda i: (0, i)),
        ],
        out_specs=[
            pl.BlockSpec((1, _S), lambda i: (0, i)),
            pl.BlockSpec((1, _S), lambda i: (0, i)),
        ],
        out_shape=[
            jax.ShapeDtypeStruct((1, _N), jnp.float32),
            jax.ShapeDtypeStruct((1, _N), jnp.float32),
        ],
        compiler_params=pltpu.CompilerParams(
            vmem_limit_bytes=100 * 1024 * 1024,
        ),
        interpret=interpret,
    )(lT, t2d)


def _seg_body(pt_hbm, lg_hbm, t_hbm, cnt_out, s1_out, s2_out, sl_out,
              pt_v, lg_v, t_v, cnt_v, s1_v, s2_v, sl_v):
    wid = lax.axis_index("s") * _NC + lax.axis_index("c")
    base = wid * _CHUNK
    pltpu.sync_copy(pt_hbm.at[pl.ds(base, _CHUNK)], pt_v)
    pltpu.sync_copy(lg_hbm.at[pl.ds(base, _CHUNK)], lg_v)
    pltpu.sync_copy(t_hbm.at[pl.ds(base, _CHUNK)], t_v)

    zeros16 = jnp.zeros((16,), jnp.float32)

    def zbody(i, carry):
        sl16 = pl.ds(i * 16, 16)
        cnt_v[sl16] = zeros16
        s1_v[sl16] = zeros16
        s2_v[sl16] = zeros16
        sl_v[sl16] = zeros16
        return carry

    lax.fori_loop(0, _CPAD // 16, zbody, 0)

    ones16 = jnp.full((16,), 1.0, jnp.float32)

    def body(i, carry):
        sl16 = pl.ds(i * 16, 16)
        t16 = t_v[sl16]
        p16 = pt_v[sl16]
        g16 = lg_v[sl16]
        plsc.addupdate_scatter(cnt_v, [t16], ones16)
        plsc.addupdate_scatter(s1_v, [t16], p16)
        plsc.addupdate_scatter(s2_v, [t16], p16 * p16)
        plsc.addupdate_scatter(sl_v, [t16], g16)
        return carry

    lax.fori_loop(0, _STEPS, body, 0)

    pltpu.sync_copy(cnt_v, cnt_out.at[wid])
    pltpu.sync_copy(s1_v, s1_out.at[wid])
    pltpu.sync_copy(s2_v, s2_out.at[wid])
    pltpu.sync_copy(sl_v, sl_out.at[wid])


@functools.cache
def _make_seg_call():
    return functools.partial(
        pl.kernel,
        mesh=plsc.VectorSubcoreMesh(core_axis_name="c", subcore_axis_name="s"),
        compiler_params=pltpu.CompilerParams(needs_layout_passes=False),
        out_type=[jax.ShapeDtypeStruct((_NW, _CPAD), jnp.float32)] * 4,
        scratch_types=[
            pltpu.VMEM((_CHUNK,), jnp.float32),
            pltpu.VMEM((_CHUNK,), jnp.float32),
            pltpu.VMEM((_CHUNK,), jnp.int32),
            pltpu.VMEM((_CPAD,), jnp.float32),
            pltpu.VMEM((_CPAD,), jnp.float32),
            pltpu.VMEM((_CPAD,), jnp.float32),
            pltpu.VMEM((_CPAD,), jnp.float32),
        ],
    )(_seg_body)


def _final_body(cnt_ref, s1_ref, s2_ref, sl_ref, out_ref):
    cnt = jnp.sum(cnt_ref[...], axis=0, keepdims=True)
    s1 = jnp.sum(s1_ref[...], axis=0, keepdims=True)
    s2 = jnp.sum(s2_ref[...], axis=0, keepdims=True)
    sl = jnp.sum(sl_ref[...], axis=0, keepdims=True)
    idx = lax.broadcasted_iota(jnp.int32, (1, _CPAD), 1)
    valid = idx < _C
    nz = cnt > 0.0
    csafe = jnp.where(nz, cnt, 1.0)
    cmax = jnp.max(cnt)
    alpha = jnp.where(nz, jnp.log(cmax / csafe) + 1.0, 0.0)
    p_avg1 = jnp.where(nz, -sl / csafe, 1.0)
    loss1 = p_avg1 * alpha
    loss1_mean = jnp.sum(jnp.where(valid, loss1, 0.0)) / _C
    mean = jnp.where(nz, s1 / csafe, 1.0)
    gt1 = cnt > 1.0
    denom = jnp.where(gt1, cnt - 1.0, 1.0)
    var = (s2 - cnt * mean * mean) / denom
    std = jnp.where(gt1, jnp.sqrt(jnp.maximum(var, 1e-12)), 0.0)
    am = jnp.max(jnp.where(valid, alpha, -1e30))
    ea = jnp.where(valid, jnp.exp(alpha - am), 0.0)
    asm = ea / jnp.sum(ea)
    loss2_c = std / mean * asm
    loss2_mean = jnp.sum(cnt * loss2_c) / _N
    out_ref[...] = jnp.full((1, 1), loss1_mean + loss2_mean, jnp.float32)


def _final_pass(cnt_p, s1_p, s2_p, sl_p, interpret=False):
    return pl.pallas_call(
        _final_body,
        out_shape=jax.ShapeDtypeStruct((1, 1), jnp.float32),
        interpret=interpret,
    )(cnt_p, s1_p, s2_p, sl_p)


@jax.jit
def kernel(logits, target):
    logits = logits.reshape(-1, _C)
    t = target.reshape(-1).astype(jnp.int32)
    pt2d, lg2d = _pt_pass(logits.T, t.reshape(1, -1))
    pt = pt2d.reshape(-1)
    lg = lg2d.reshape(-1)
    cnt_p, s1_p, s2_p, sl_p = _make_seg_call()(pt, lg, t)
    out = _final_pass(cnt_p, s1_p, s2_p, sl_p)
    return out[0, 0]


# speedup vs baseline: 7.2659x; 1.0141x over previous
import functools

import jax
import jax.numpy as jnp
from jax import lax
from jax.experimental import pallas as pl
from jax.experimental.pallas import tpu as pltpu
from jax.experimental.pallas import tpu_sc as plsc

_N = 65536
_C = 1000
_CPAD = 1024
_S = 4096
_NB = _N // _S
_NC = 2
_NS = 16
_NW = _NC * _NS
_CHUNK = _N // _NW
_STEPS = _CHUNK // 16


def _pt_body(l_ref, t_ref, pt_ref, lg_ref):
    l = l_ref[...]
    t = t_ref[...]
    m = jnp.max(l, axis=0, keepdims=True)
    s = jnp.sum(jnp.exp(l - m), axis=0, keepdims=True)
    rows = lax.broadcasted_iota(jnp.int32, (_C, _S), 0)
    lt = jnp.sum(jnp.where(rows == t, l, 0.0), axis=0, keepdims=True)
    pt = jnp.exp(lt - m) / s
    pt_ref[...] = pt
    lg_ref[...] = jnp.log(pt + 1e-6)


def _pt_pass(lT, t2d, interpret=False):
    return pl.pallas_call(
        _pt_body,
        grid=(_NB,),
        in_specs=[
            pl.BlockSpec((_C, _S), lambda i: (0, i)),
            pl.BlockSpec((1, _S), lambda i: (0, i)),
        ],
        out_specs=[
            pl.BlockSpec((1, _S), lambda i: (0, i)),
            pl.BlockSpec((1, _S), lambda i: (0, i)),
        ],
        out_shape=[
            jax.ShapeDtypeStruct((1, _N), jnp.float32),
            jax.ShapeDtypeStruct((1, _N), jnp.float32),
        ],
        compiler_params=pltpu.CompilerParams(
            vmem_limit_bytes=100 * 1024 * 1024,
        ),
        interpret=interpret,
    )(lT, t2d)


def _seg_body(pt_hbm, lg_hbm, t_hbm, cnt_out, s1_out, s2_out, sl_out,
              pt_v, lg_v, t_v, cnt_v, s1_v, s2_v, sl_v, sem_in, sem_out):
    wid = lax.axis_index("s") * _NC + lax.axis_index("c")
    base = wid * _CHUNK
    cp_pt = pltpu.async_copy(pt_hbm.at[pl.ds(base, _CHUNK)], pt_v, sem_in)
    cp_lg = pltpu.async_copy(lg_hbm.at[pl.ds(base, _CHUNK)], lg_v, sem_in)
    cp_t = pltpu.async_copy(t_hbm.at[pl.ds(base, _CHUNK)], t_v, sem_in)

    zeros16 = jnp.zeros((16,), jnp.float32)

    def zbody(i, carry):
        sl16 = pl.ds(i * 16, 16)
        cnt_v[sl16] = zeros16
        s1_v[sl16] = zeros16
        s2_v[sl16] = zeros16
        sl_v[sl16] = zeros16
        return carry

    lax.fori_loop(0, _CPAD // 16, zbody, 0)
    cp_pt.wait()
    cp_lg.wait()
    cp_t.wait()

    ones16 = jnp.full((16,), 1.0, jnp.float32)

    def body(i, carry):
        sl16 = pl.ds(i * 16, 16)
        t16 = t_v[sl16]
        p16 = pt_v[sl16]
        g16 = lg_v[sl16]
        plsc.addupdate_scatter(cnt_v, [t16], ones16)
        plsc.addupdate_scatter(s1_v, [t16], p16)
        plsc.addupdate_scatter(s2_v, [t16], p16 * p16)
        plsc.addupdate_scatter(sl_v, [t16], g16)
        return carry

    lax.fori_loop(0, _STEPS, body, 0)

    wr_cnt = pltpu.async_copy(cnt_v, cnt_out.at[wid], sem_out)
    wr_s1 = pltpu.async_copy(s1_v, s1_out.at[wid], sem_out)
    wr_s2 = pltpu.async_copy(s2_v, s2_out.at[wid], sem_out)
    wr_sl = pltpu.async_copy(sl_v, sl_out.at[wid], sem_out)
    wr_cnt.wait()
    wr_s1.wait()
    wr_s2.wait()
    wr_sl.wait()


@functools.cache
def _make_seg_call():
    return functools.partial(
        pl.kernel,
        mesh=plsc.VectorSubcoreMesh(core_axis_name="c", subcore_axis_name="s"),
        compiler_params=pltpu.CompilerParams(needs_layout_passes=False),
        out_type=[jax.ShapeDtypeStruct((_NW, _CPAD), jnp.float32)] * 4,
        scratch_types=[
            pltpu.VMEM((_CHUNK,), jnp.float32),
            pltpu.VMEM((_CHUNK,), jnp.float32),
            pltpu.VMEM((_CHUNK,), jnp.int32),
            pltpu.VMEM((_CPAD,), jnp.float32),
            pltpu.VMEM((_CPAD,), jnp.float32),
            pltpu.VMEM((_CPAD,), jnp.float32),
            pltpu.VMEM((_CPAD,), jnp.float32),
            pltpu.SemaphoreType.DMA,
            pltpu.SemaphoreType.DMA,
        ],
    )(_seg_body)


def _final_body(cnt_ref, s1_ref, s2_ref, sl_ref, out_ref):
    cnt = jnp.sum(cnt_ref[...], axis=0, keepdims=True)
    s1 = jnp.sum(s1_ref[...], axis=0, keepdims=True)
    s2 = jnp.sum(s2_ref[...], axis=0, keepdims=True)
    sl = jnp.sum(sl_ref[...], axis=0, keepdims=True)
    idx = lax.broadcasted_iota(jnp.int32, (1, _CPAD), 1)
    valid = idx < _C
    nz = cnt > 0.0
    csafe = jnp.where(nz, cnt, 1.0)
    cmax = jnp.max(cnt)
    alpha = jnp.where(nz, jnp.log(cmax / csafe) + 1.0, 0.0)
    p_avg1 = jnp.where(nz, -sl / csafe, 1.0)
    loss1 = p_avg1 * alpha
    loss1_mean = jnp.sum(jnp.where(valid, loss1, 0.0)) / _C
    mean = jnp.where(nz, s1 / csafe, 1.0)
    gt1 = cnt > 1.0
    denom = jnp.where(gt1, cnt - 1.0, 1.0)
    var = (s2 - cnt * mean * mean) / denom
    std = jnp.where(gt1, jnp.sqrt(jnp.maximum(var, 1e-12)), 0.0)
    am = jnp.max(jnp.where(valid, alpha, -1e30))
    ea = jnp.where(valid, jnp.exp(alpha - am), 0.0)
    asm = ea / jnp.sum(ea)
    loss2_c = std / mean * asm
    loss2_mean = jnp.sum(cnt * loss2_c) / _N
    out_ref[...] = jnp.full((1, 1), loss1_mean + loss2_mean, jnp.float32)


def _final_pass(cnt_p, s1_p, s2_p, sl_p, interpret=False):
    return pl.pallas_call(
        _final_body,
        out_shape=jax.ShapeDtypeStruct((1, 1), jnp.float32),
        interpret=interpret,
    )(cnt_p, s1_p, s2_p, sl_p)


@jax.jit
def kernel(logits, target):
    logits = logits.reshape(-1, _C)
    t = target.reshape(-1).astype(jnp.int32)
    pt2d, lg2d = _pt_pass(logits.T, t.reshape(1, -1))
    pt = pt2d.reshape(-1)
    lg = lg2d.reshape(-1)
    cnt_p, s1_p, s2_p, sl_p = _make_seg_call()(pt, lg, t)
    out = _final_pass(cnt_p, s1_p, s2_p, sl_p)
    return out[0, 0]
